# baseline (speedup 1.0000x reference)
import functools

import jax
import jax.numpy as jnp
from jax import lax
from jax.experimental import pallas as pl
from jax.experimental.pallas import tpu as pltpu
from jax.experimental.pallas import tpu_sc as plsc

N = 8192
NF = 16384
C = 256
BQ = 1024
NBLK = NF // BQ
CB = 128
NCHUNK = N // CB
ALIGN = NCHUNK // NBLK

GCHUNK = 128

_OFFS = list(range(ALIGN))
for _k in range(1, NCHUNK):
    if len(_OFFS) < NCHUNK:
        _OFFS.append(ALIGN - 1 + _k)
    if len(_OFFS) < NCHUNK:
        _OFFS.append(-_k)
assert len(_OFFS) == NCHUNK and len(set(o % NCHUNK for o in _OFFS)) == NCHUNK


def _nn_body(qbnd_ref, cbnd_ref, q_ref, ct_ref, perm_ref, out_ref, rm_ref, ri_ref):
    i = pl.program_id(1)
    q0 = q_ref[0, :, 0:1]
    q1 = q_ref[0, :, 1:2]
    q2 = q_ref[0, :, 2:3]
    x0 = qbnd_ref[0, 0, 2 * i]
    x1 = qbnd_ref[0, 0, 2 * i + 1]
    rm_ref[...] = jnp.full((BQ, CB), jnp.inf, jnp.float32)
    ri_ref[...] = jnp.zeros((BQ, CB), jnp.int32)

    def eval_chunk(c):
        chunk = ct_ref[0, c]
        pc = perm_ref[0, c]
        d0 = q0 - chunk[0:1, :]
        d1 = q1 - chunk[1:2, :]
        d2 = q2 - chunk[2:3, :]
        dc = (d0 * d0 + d1 * d1) + d2 * d2
        rm = rm_ref[...]
        better = dc < rm
        rm_ref[...] = jnp.minimum(rm, dc)
        ri_ref[...] = jnp.where(better, jnp.broadcast_to(pc, (BQ, CB)), ri_ref[...])

    ub = None
    for j, off in enumerate(_OFFS):
        c = lax.rem(i * ALIGN + off + NCHUNK, NCHUNK)
        if j < ALIGN:
            eval_chunk(c)
            continue
        if (j - ALIGN) % 8 == 0:
            ub = jnp.max(jnp.min(rm_ref[...], axis=1, keepdims=True)) * 1.00001
        cx0 = cbnd_ref[0, 0, 2 * c]
        cx1 = cbnd_ref[0, 0, 2 * c + 1]
        lb = jnp.maximum(jnp.maximum(x0 - cx1, cx0 - x1), 0.0)
        pl.when(lb * lb <= ub)(lambda c=c: eval_chunk(c))

    rm = rm_ref[...]
    m = jnp.min(rm, axis=1, keepdims=True)
    idx = jnp.min(jnp.where(rm == m, ri_ref[...], N), axis=1)
    out_ref[0, 0, 0, :] = idx


def _nn_indices(qbnd, cbnd, qs, ct4, perm4):
    return pl.pallas_call(
        _nn_body,
        grid=(2, NBLK),
        in_specs=[
            pl.BlockSpec((1, 1, 2 * NBLK), lambda s, i: (s, 0, 0),
                         memory_space=pltpu.SMEM),
            pl.BlockSpec((1, 1, 2 * NCHUNK), lambda s, i: (s, 0, 0),
                         memory_space=pltpu.SMEM),
            pl.BlockSpec((1, BQ, 3), lambda s, i: (s, i, 0)),
            pl.BlockSpec((1, NCHUNK, 3, CB), lambda s, i: (s, 0, 0, 0)),
            pl.BlockSpec((1, NCHUNK, 1, CB), lambda s, i: (s, 0, 0, 0)),
        ],
        out_specs=pl.BlockSpec((1, 1, 1, BQ), lambda s, i: (s, i, 0, 0)),
        out_shape=jax.ShapeDtypeStruct((2, NBLK, 1, BQ), jnp.int32),
        scratch_shapes=[
            pltpu.VMEM((BQ, CB), jnp.float32),
            pltpu.VMEM((BQ, CB), jnp.int32),
        ],
    )(qbnd, cbnd, qs, ct4, perm4)


def _sc_gather_concat(src, tgt, idx3, scat3):
    info = plsc.get_sparse_core_info()
    nw = info.num_cores * info.num_subcores
    copy_pw = N // nw
    gath_pw = NF // nw
    nchunk = gath_pw // GCHUNK

    cchunk = copy_pw // GCHUNK
    jside = nchunk + cchunk
    njobs = 2 * jside
    NBUF = 3

    mesh = plsc.VectorSubcoreMesh(core_axis_name="c", subcore_axis_name="s")

    @functools.partial(
        pl.kernel,
        mesh=mesh,
        out_type=[
            jax.ShapeDtypeStruct((N + NF, C), jnp.float32),
            jax.ShapeDtypeStruct((N + NF, C), jnp.float32),
        ],
        scratch_types=[
            pltpu.VMEM((2, nchunk, GCHUNK), jnp.int32),
            pltpu.VMEM((2, nchunk, GCHUNK), jnp.int32),
        ]
        + [pltpu.VMEM((GCHUNK, C), jnp.float32)] * NBUF
        + [pltpu.SemaphoreType.DMA] * (2 * NBUF + 1),
    )
    def sc_k(src_hbm, tgt_hbm, idx_hbm, scat_hbm, src_out, tgt_out,
             idx_v, scat_v, *rest):
        bufs = rest[:NBUF]
        gsem = rest[NBUF:2 * NBUF]
        ssem = rest[2 * NBUF:3 * NBUF]
        isem = rest[3 * NBUF]
        wid = lax.axis_index("s") * info.num_cores + lax.axis_index("c")
        tabs = (src_hbm, tgt_hbm)
        outs = (src_out, tgt_out)
        gb = wid * nchunk
        cb = wid * copy_pw

        pltpu.async_copy(idx_hbm.at[0, pl.ds(gb, nchunk)], idx_v.at[0], isem).wait()
        pltpu.async_copy(idx_hbm.at[1, pl.ds(gb, nchunk)], idx_v.at[1], isem).wait()
        pltpu.async_copy(scat_hbm.at[0, pl.ds(gb, nchunk)], scat_v.at[0], isem).wait()
        pltpu.async_copy(scat_hbm.at[1, pl.ds(gb, nchunk)], scat_v.at[1], isem).wait()

        gcp = [None] * njobs
        scp = [None] * njobs

        def fire(g):
            s, k = divmod(g, jside)
            if k < nchunk:
                src_ref = tabs[s].at[idx_v.at[s, k]]
            else:
                src_ref = tabs[s].at[pl.ds(cb + (k - nchunk) * GCHUNK, GCHUNK)]
            gcp[g] = pltpu.async_copy(src_ref, bufs[g % NBUF], gsem[g % NBUF])

        def store(g):
            s, k = divmod(g, jside)
            if k < nchunk:
                dst_ref = outs[s].at[scat_v.at[s, k]]
            else:
                dst_ref = outs[s].at[pl.ds(cb + (k - nchunk) * GCHUNK, GCHUNK)]
            scp[g] = pltpu.async_copy(bufs[g % NBUF], dst_ref, ssem[g % NBUF])

        for g in range(min(NBUF, njobs)):
            fire(g)
        for g in range(njobs):
            gcp[g].wait()
            store(g)
            if g + NBUF < njobs:
                scp[g].wait()
                fire(g + NBUF)
        for g in range(max(0, njobs - NBUF), njobs):
            scp[g].wait()

    return sc_k(src, tgt, idx3, scat3)


def kernel(src, tgt, src_coords, tgt_coords, src_shortcut, tgt_shortcut,
           src_shortcut_coords, tgt_shortcut_coords):
    coords2 = jnp.stack([src_coords, tgt_coords])
    cx, corder, cy, cz = lax.sort(
        (coords2[:, :, 0],
         jnp.broadcast_to(jnp.arange(N, dtype=jnp.int32), (2, N)),
         coords2[:, :, 1], coords2[:, :, 2]), dimension=1, num_keys=1)
    ct4 = jnp.stack([cx, cy, cz], axis=1).reshape(2, 3, NCHUNK, CB).transpose(0, 2, 1, 3)
    perm4 = corder.reshape(2, NCHUNK, 1, CB)
    cxc = cx.reshape(2, NCHUNK, CB)
    cbnd = jnp.stack([cxc[:, :, 0], cxc[:, :, -1]], axis=-1).reshape(2, 1, 2 * NCHUNK)

    q2 = jnp.stack([src_shortcut_coords, tgt_shortcut_coords])
    qx, qorder, qy, qz = lax.sort(
        (q2[:, :, 0],
         jnp.broadcast_to(jnp.arange(NF, dtype=jnp.int32), (2, NF)),
         q2[:, :, 1], q2[:, :, 2]), dimension=1, num_keys=1)
    qs = jnp.stack([qx, qy, qz], axis=-1)
    qxb = qx.reshape(2, NBLK, BQ)
    qbnd = jnp.stack([qxb[:, :, 0], qxb[:, :, -1]], axis=-1).reshape(2, 1, 2 * NBLK)
    scat3 = (qorder + N).reshape(2, NF // GCHUNK, GCHUNK)

    idx = _nn_indices(qbnd, cbnd, qs, ct4, perm4)
    idx3 = idx.reshape(2, NF // GCHUNK, GCHUNK)
    src_out, tgt_out = _sc_gather_concat(src, tgt, idx3, scat3)
    return (src_out, tgt_out, src_shortcut_coords, tgt_shortcut_coords)


# speedup vs baseline: 1.2690x; 1.2690x over previous
import functools

import jax
import jax.numpy as jnp
from jax import lax
from jax.experimental import pallas as pl
from jax.experimental.pallas import tpu as pltpu
from jax.experimental.pallas import tpu_sc as plsc

N = 8192
NF = 16384
C = 256
BQ = 1024
NBLK = NF // BQ
CB = 128
NCHUNK = N // CB
ALIGN = NCHUNK // NBLK

GCHUNK = 128

_OFFS = list(range(ALIGN))
for _k in range(1, NCHUNK):
    if len(_OFFS) < NCHUNK:
        _OFFS.append(ALIGN - 1 + _k)
    if len(_OFFS) < NCHUNK:
        _OFFS.append(-_k)
assert len(_OFFS) == NCHUNK and len(set(o % NCHUNK for o in _OFFS)) == NCHUNK


def _nn_body(qbnd_ref, cbnd_ref, q_ref, ct_ref, perm_ref, out_ref, rm_ref, ri_ref):
    i = pl.program_id(1)
    q0 = q_ref[0, :, 0:1]
    q1 = q_ref[0, :, 1:2]
    q2 = q_ref[0, :, 2:3]
    x0 = qbnd_ref[0, 0, 2 * i]
    x1 = qbnd_ref[0, 0, 2 * i + 1]
    rm_ref[...] = jnp.full((BQ, CB), jnp.inf, jnp.float32)
    ri_ref[...] = jnp.zeros((BQ, CB), jnp.int32)

    def eval_chunk(c):
        chunk = ct_ref[0, c]
        pc = perm_ref[0, c]
        d0 = q0 - chunk[0:1, :]
        d1 = q1 - chunk[1:2, :]
        d2 = q2 - chunk[2:3, :]
        dc = (d0 * d0 + d1 * d1) + d2 * d2
        rm = rm_ref[...]
        better = dc < rm
        rm_ref[...] = jnp.minimum(rm, dc)
        ri_ref[...] = jnp.where(better, jnp.broadcast_to(pc, (BQ, CB)), ri_ref[...])

    ub = None
    for j, off in enumerate(_OFFS):
        c = lax.rem(i * ALIGN + off + NCHUNK, NCHUNK)
        if j < ALIGN:
            eval_chunk(c)
            continue
        if (j - ALIGN) % 8 == 0:
            ub = jnp.max(jnp.min(rm_ref[...], axis=1, keepdims=True)) * 1.00001
        cx0 = cbnd_ref[0, 0, 2 * c]
        cx1 = cbnd_ref[0, 0, 2 * c + 1]
        lb = jnp.maximum(jnp.maximum(x0 - cx1, cx0 - x1), 0.0)
        pl.when(lb * lb <= ub)(lambda c=c: eval_chunk(c))

    rm = rm_ref[...]
    m = jnp.min(rm, axis=1, keepdims=True)
    idx = jnp.min(jnp.where(rm == m, ri_ref[...], N), axis=1)
    out_ref[0, 0, 0, :] = idx


def _nn_indices(qbnd, cbnd, qs, ct4, perm4):
    return pl.pallas_call(
        _nn_body,
        grid=(2, NBLK),
        in_specs=[
            pl.BlockSpec((1, 1, 2 * NBLK), lambda s, i: (s, 0, 0),
                         memory_space=pltpu.SMEM),
            pl.BlockSpec((1, 1, 2 * NCHUNK), lambda s, i: (s, 0, 0),
                         memory_space=pltpu.SMEM),
            pl.BlockSpec((1, BQ, 3), lambda s, i: (s, i, 0)),
            pl.BlockSpec((1, NCHUNK, 3, CB), lambda s, i: (s, 0, 0, 0)),
            pl.BlockSpec((1, NCHUNK, 1, CB), lambda s, i: (s, 0, 0, 0)),
        ],
        out_specs=pl.BlockSpec((1, 1, 1, BQ), lambda s, i: (s, i, 0, 0)),
        out_shape=jax.ShapeDtypeStruct((2, NBLK, 1, BQ), jnp.int32),
        scratch_shapes=[
            pltpu.VMEM((BQ, CB), jnp.float32),
            pltpu.VMEM((BQ, CB), jnp.int32),
        ],
    )(qbnd, cbnd, qs, ct4, perm4)


def _sc_gather_concat(src, tgt, idx3, scat3):
    info = plsc.get_sparse_core_info()
    nw = info.num_cores * info.num_subcores
    copy_pw = N // nw
    gath_pw = NF // nw
    nchunk = gath_pw // GCHUNK

    cchunk = copy_pw // GCHUNK
    jside = nchunk + cchunk
    njobs = 2 * jside
    NBUF = 3

    mesh = plsc.VectorSubcoreMesh(core_axis_name="c", subcore_axis_name="s")

    @functools.partial(
        pl.kernel,
        mesh=mesh,
        out_type=[
            jax.ShapeDtypeStruct((N + NF, C), jnp.float32),
            jax.ShapeDtypeStruct((N + NF, C), jnp.float32),
        ],
        scratch_types=[
            pltpu.VMEM((2, nchunk, GCHUNK), jnp.int32),
            pltpu.VMEM((2, nchunk, GCHUNK), jnp.int32),
        ]
        + [pltpu.VMEM((GCHUNK, C), jnp.float32)] * NBUF
        + [pltpu.SemaphoreType.DMA] * (2 * NBUF + 1),
    )
    def sc_k(src_hbm, tgt_hbm, idx_hbm, scat_hbm, src_out, tgt_out,
             idx_v, scat_v, *rest):
        bufs = rest[:NBUF]
        gsem = rest[NBUF:2 * NBUF]
        ssem = rest[2 * NBUF:3 * NBUF]
        isem = rest[3 * NBUF]
        wid = lax.axis_index("s") * info.num_cores + lax.axis_index("c")
        tabs = (src_hbm, tgt_hbm)
        outs = (src_out, tgt_out)
        gb = wid * nchunk
        cb = wid * copy_pw

        pltpu.async_copy(idx_hbm.at[0, pl.ds(gb, nchunk)], idx_v.at[0], isem).wait()
        pltpu.async_copy(idx_hbm.at[1, pl.ds(gb, nchunk)], idx_v.at[1], isem).wait()
        pltpu.async_copy(scat_hbm.at[0, pl.ds(gb, nchunk)], scat_v.at[0], isem).wait()
        pltpu.async_copy(scat_hbm.at[1, pl.ds(gb, nchunk)], scat_v.at[1], isem).wait()

        gcp = [None] * njobs
        scp = [None] * njobs

        def fire(g):
            s, k = divmod(g, jside)
            if k < nchunk:
                src_ref = tabs[s].at[idx_v.at[s, k]]
            else:
                src_ref = tabs[s].at[pl.ds(cb + (k - nchunk) * GCHUNK, GCHUNK)]
            gcp[g] = pltpu.async_copy(src_ref, bufs[g % NBUF], gsem[g % NBUF])

        def store(g):
            s, k = divmod(g, jside)
            if k < nchunk:
                dst_ref = outs[s].at[scat_v.at[s, k]]
            else:
                dst_ref = outs[s].at[pl.ds(cb + (k - nchunk) * GCHUNK, GCHUNK)]
            scp[g] = pltpu.async_copy(bufs[g % NBUF], dst_ref, ssem[g % NBUF])

        for g in range(min(NBUF, njobs)):
            fire(g)
        for g in range(njobs):
            gcp[g].wait()
            store(g)
            if g + NBUF < njobs:
                scp[g].wait()
                fire(g + NBUF)
        for g in range(max(0, njobs - NBUF), njobs):
            scp[g].wait()

    return sc_k(src, tgt, idx3, scat3)


def kernel(src, tgt, src_coords, tgt_coords, src_shortcut, tgt_shortcut,
           src_shortcut_coords, tgt_shortcut_coords):
    ct4s, perm4s, cbnds = [], [], []
    for coords in (src_coords, tgt_coords):
        cx, corder, cy, cz = lax.sort(
            (coords[:, 0], jnp.arange(N, dtype=jnp.int32),
             coords[:, 1], coords[:, 2]), num_keys=1)
        ct4s.append(jnp.stack([cx, cy, cz]).reshape(3, NCHUNK, CB).transpose(1, 0, 2))
        perm4s.append(corder.reshape(NCHUNK, 1, CB))
        cxc = cx.reshape(NCHUNK, CB)
        cbnds.append(jnp.stack([cxc[:, 0], cxc[:, -1]], axis=-1).reshape(1, 2 * NCHUNK))
    qss, qbnds, scats = [], [], []
    for q in (src_shortcut_coords, tgt_shortcut_coords):
        qx, qorder, qy, qz = lax.sort(
            (q[:, 0], jnp.arange(NF, dtype=jnp.int32),
             q[:, 1], q[:, 2]), num_keys=1)
        qss.append(jnp.stack([qx, qy, qz], axis=-1))
        qxb = qx.reshape(NBLK, BQ)
        qbnds.append(jnp.stack([qxb[:, 0], qxb[:, -1]], axis=-1).reshape(1, 2 * NBLK))
        scats.append((qorder + N).reshape(NF // GCHUNK, GCHUNK))

    idx = _nn_indices(jnp.stack(qbnds), jnp.stack(cbnds), jnp.stack(qss),
                      jnp.stack(ct4s), jnp.stack(perm4s))
    idx3 = idx.reshape(2, NF // GCHUNK, GCHUNK)
    src_out, tgt_out = _sc_gather_concat(src, tgt, idx3, jnp.stack(scats))
    return (src_out, tgt_out, src_shortcut_coords, tgt_shortcut_coords)


# speedup vs baseline: 1.3418x; 1.0573x over previous
import functools

import jax
import jax.numpy as jnp
from jax import lax
from jax.experimental import pallas as pl
from jax.experimental.pallas import tpu as pltpu
from jax.experimental.pallas import tpu_sc as plsc

N = 8192
NF = 16384
C = 256
BQ = 1024
NBLK = NF // BQ
CB = 128
NCHUNK = N // CB
ALIGN = NCHUNK // NBLK

GCHUNK = 128

_OFFS = list(range(ALIGN))
for _k in range(1, NCHUNK):
    if len(_OFFS) < NCHUNK:
        _OFFS.append(ALIGN - 1 + _k)
    if len(_OFFS) < NCHUNK:
        _OFFS.append(-_k)
assert len(_OFFS) == NCHUNK and len(set(o % NCHUNK for o in _OFFS)) == NCHUNK


def _nn_body(qbnd_ref, cbnd_ref, q_ref, ct_ref, perm_ref, out_ref, rm_ref, ri_ref):
    i = pl.program_id(1)
    q0 = q_ref[0, :, 0:1]
    q1 = q_ref[0, :, 1:2]
    q2 = q_ref[0, :, 2:3]
    x0 = qbnd_ref[0, 0, 2 * i]
    x1 = qbnd_ref[0, 0, 2 * i + 1]
    rm_ref[...] = jnp.full((BQ, CB), jnp.inf, jnp.float32)
    ri_ref[...] = jnp.zeros((BQ, CB), jnp.int32)

    def eval_chunk(c):
        chunk = ct_ref[0, c]
        pc = perm_ref[0, c]
        d0 = q0 - chunk[0:1, :]
        d1 = q1 - chunk[1:2, :]
        d2 = q2 - chunk[2:3, :]
        dc = (d0 * d0 + d1 * d1) + d2 * d2
        rm = rm_ref[...]
        better = dc < rm
        rm_ref[...] = jnp.minimum(rm, dc)
        ri_ref[...] = jnp.where(better, jnp.broadcast_to(pc, (BQ, CB)), ri_ref[...])

    ub = None
    for j, off in enumerate(_OFFS):
        c = lax.rem(i * ALIGN + off + NCHUNK, NCHUNK)
        if j < ALIGN:
            eval_chunk(c)
            continue
        if (j - ALIGN) % 16 == 0:
            ub = jnp.max(jnp.min(rm_ref[...], axis=1, keepdims=True)) * 1.00001
        cx0 = cbnd_ref[0, 0, 2 * c]
        cx1 = cbnd_ref[0, 0, 2 * c + 1]
        lb = jnp.maximum(jnp.maximum(x0 - cx1, cx0 - x1), 0.0)
        pl.when(lb * lb <= ub)(lambda c=c: eval_chunk(c))

    rm = rm_ref[...]
    m = jnp.min(rm, axis=1, keepdims=True)
    idx = jnp.min(jnp.where(rm == m, ri_ref[...], N), axis=1)
    out_ref[0, 0, 0, :] = idx


def _nn_indices(qbnd, cbnd, qs, ct4, perm4):
    return pl.pallas_call(
        _nn_body,
        grid=(2, NBLK),
        in_specs=[
            pl.BlockSpec((1, 1, 2 * NBLK), lambda s, i: (s, 0, 0),
                         memory_space=pltpu.SMEM),
            pl.BlockSpec((1, 1, 2 * NCHUNK), lambda s, i: (s, 0, 0),
                         memory_space=pltpu.SMEM),
            pl.BlockSpec((1, BQ, 3), lambda s, i: (s, i, 0)),
            pl.BlockSpec((1, NCHUNK, 3, CB), lambda s, i: (s, 0, 0, 0)),
            pl.BlockSpec((1, NCHUNK, 1, CB), lambda s, i: (s, 0, 0, 0)),
        ],
        out_specs=pl.BlockSpec((1, 1, 1, BQ), lambda s, i: (s, i, 0, 0)),
        out_shape=jax.ShapeDtypeStruct((2, NBLK, 1, BQ), jnp.int32),
        scratch_shapes=[
            pltpu.VMEM((BQ, CB), jnp.float32),
            pltpu.VMEM((BQ, CB), jnp.int32),
        ],
    )(qbnd, cbnd, qs, ct4, perm4)


def _sc_gather_concat(src, tgt, idx3, scat3):
    info = plsc.get_sparse_core_info()
    nw = info.num_cores * info.num_subcores
    copy_pw = N // nw
    gath_pw = NF // nw
    nchunk = gath_pw // GCHUNK

    cchunk = copy_pw // GCHUNK
    jside = nchunk + cchunk
    njobs = 2 * jside
    NBUF = 3

    mesh = plsc.VectorSubcoreMesh(core_axis_name="c", subcore_axis_name="s")

    @functools.partial(
        pl.kernel,
        mesh=mesh,
        out_type=[
            jax.ShapeDtypeStruct((N + NF, C), jnp.float32),
            jax.ShapeDtypeStruct((N + NF, C), jnp.float32),
        ],
        scratch_types=[
            pltpu.VMEM((2, nchunk, GCHUNK), jnp.int32),
            pltpu.VMEM((2, nchunk, GCHUNK), jnp.int32),
        ]
        + [pltpu.VMEM((GCHUNK, C), jnp.float32)] * NBUF
        + [pltpu.SemaphoreType.DMA] * (2 * NBUF + 1),
    )
    def sc_k(src_hbm, tgt_hbm, idx_hbm, scat_hbm, src_out, tgt_out,
             idx_v, scat_v, *rest):
        bufs = rest[:NBUF]
        gsem = rest[NBUF:2 * NBUF]
        ssem = rest[2 * NBUF:3 * NBUF]
        isem = rest[3 * NBUF]
        wid = lax.axis_index("s") * info.num_cores + lax.axis_index("c")
        tabs = (src_hbm, tgt_hbm)
        outs = (src_out, tgt_out)
        gb = wid * nchunk
        cb = wid * copy_pw

        pltpu.async_copy(idx_hbm.at[0, pl.ds(gb, nchunk)], idx_v.at[0], isem).wait()
        pltpu.async_copy(idx_hbm.at[1, pl.ds(gb, nchunk)], idx_v.at[1], isem).wait()
        pltpu.async_copy(scat_hbm.at[0, pl.ds(gb, nchunk)], scat_v.at[0], isem).wait()
        pltpu.async_copy(scat_hbm.at[1, pl.ds(gb, nchunk)], scat_v.at[1], isem).wait()

        gcp = [None] * njobs
        scp = [None] * njobs

        def fire(g):
            s, k = divmod(g, jside)
            if k < nchunk:
                src_ref = tabs[s].at[idx_v.at[s, k]]
            else:
                src_ref = tabs[s].at[pl.ds(cb + (k - nchunk) * GCHUNK, GCHUNK)]
            gcp[g] = pltpu.async_copy(src_ref, bufs[g % NBUF], gsem[g % NBUF])

        def store(g):
            s, k = divmod(g, jside)
            if k < nchunk:
                dst_ref = outs[s].at[scat_v.at[s, k]]
            else:
                dst_ref = outs[s].at[pl.ds(cb + (k - nchunk) * GCHUNK, GCHUNK)]
            scp[g] = pltpu.async_copy(bufs[g % NBUF], dst_ref, ssem[g % NBUF])

        for g in range(min(NBUF, njobs)):
            fire(g)
        for g in range(njobs):
            gcp[g].wait()
            store(g)
            if g + NBUF < njobs:
                scp[g].wait()
                fire(g + NBUF)
        for g in range(max(0, njobs - NBUF), njobs):
            scp[g].wait()

    return sc_k(src, tgt, idx3, scat3)


def kernel(src, tgt, src_coords, tgt_coords, src_shortcut, tgt_shortcut,
           src_shortcut_coords, tgt_shortcut_coords):
    ct4s, perm4s, cbnds = [], [], []
    for coords in (src_coords, tgt_coords):
        cx, corder, cy, cz = lax.sort(
            (coords[:, 0], jnp.arange(N, dtype=jnp.int32),
             coords[:, 1], coords[:, 2]), num_keys=1)
        ct4s.append(jnp.stack([cx, cy, cz]).reshape(3, NCHUNK, CB).transpose(1, 0, 2))
        perm4s.append(corder.reshape(NCHUNK, 1, CB))
        cxc = cx.reshape(NCHUNK, CB)
        cbnds.append(jnp.stack([cxc[:, 0], cxc[:, -1]], axis=-1).reshape(1, 2 * NCHUNK))
    qss, qbnds, scats = [], [], []
    for q in (src_shortcut_coords, tgt_shortcut_coords):
        qx, qorder, qy, qz = lax.sort(
            (q[:, 0], jnp.arange(NF, dtype=jnp.int32),
             q[:, 1], q[:, 2]), num_keys=1)
        qss.append(jnp.stack([qx, qy, qz], axis=-1))
        qxb = qx.reshape(NBLK, BQ)
        qbnds.append(jnp.stack([qxb[:, 0], qxb[:, -1]], axis=-1).reshape(1, 2 * NBLK))
        scats.append((qorder + N).reshape(NF // GCHUNK, GCHUNK))

    idx = _nn_indices(jnp.stack(qbnds), jnp.stack(cbnds), jnp.stack(qss),
                      jnp.stack(ct4s), jnp.stack(perm4s))
    idx3 = idx.reshape(2, NF // GCHUNK, GCHUNK)
    src_out, tgt_out = _sc_gather_concat(src, tgt, idx3, jnp.stack(scats))
    return (src_out, tgt_out, src_shortcut_coords, tgt_shortcut_coords)
